# Initial kernel scaffold; baseline (speedup 1.0000x reference)
import jax
import jax.numpy as jnp
from jax.experimental import pallas as pl


def kernel(loss, Tb, i_current_epoch):
    raise NotImplementedError("write your pallas kernel here")


# speedup vs baseline: 4.6078x; 4.6078x over previous
import jax
import jax.numpy as jnp
from jax import lax
from jax.experimental import pallas as pl
from jax.experimental.pallas import tpu as pltpu
from jax.experimental.pallas import tpu_sc as plsc

N_CLASSES = 1000
BS = 16384
NBINS = 1024
L = 16
NSUB = 16
NW = 32
HCHUNK = BS // NSUB
OCHUNK = BS // NW


def _body(loss_hbm, tb_hbm, ep_hbm, out_hbm,
          tb_v, hist, colblk, cnt64, cnt_v, tb_o, loss_v, out_v, ep_v,
          shared, counts_sh):
    s = lax.axis_index("s")
    c = lax.axis_index("c")
    wid = s * 2 + c

    ones = jnp.full((L,), 1.0, jnp.float32)
    zeros = jnp.zeros((L,), jnp.float32)

    pltpu.sync_copy(tb_hbm.at[pl.ds(s * HCHUNK, HCHUNK)], tb_v)
    pltpu.sync_copy(ep_hbm, ep_v)
    for k in range(NBINS // L):
        hist[pl.ds(k * L, L)] = zeros
    for k in range(HCHUNK // L):
        idx = tb_v[pl.ds(k * L, L)]
        plsc.addupdate_scatter(hist, [idx], ones)
    pltpu.sync_copy(hist, shared.at[pl.ds(s * NBINS, NBINS)])
    plsc.subcore_barrier()

    for r in range(NSUB):
        pltpu.sync_copy(shared.at[pl.ds(r * NBINS + s * 64, 64)],
                        colblk.at[pl.ds(r * 64, 64)])
    acc = [jnp.zeros((L,), jnp.float32) for _ in range(4)]
    for r in range(NSUB):
        for v in range(4):
            acc[v] = acc[v] + colblk[pl.ds(r * 64 + v * L, L)]
    for v in range(4):
        cnt64[pl.ds(v * L, L)] = acc[v]
    pltpu.sync_copy(cnt64, counts_sh.at[pl.ds(s * 64, 64)])
    plsc.subcore_barrier()

    pltpu.sync_copy(counts_sh, cnt_v)
    wsum = jnp.zeros((L,), jnp.float32)
    inv_total = jnp.float32(1.0 / BS)
    for k in range(NBINS // L):
        cv = cnt_v[pl.ds(k * L, L)]
        pos = cv > 0.0
        prob = jnp.where(pos, cv * inv_total, ones)
        rv = jnp.where(pos, 1.0 / prob, zeros)
        hist[pl.ds(k * L, L)] = rv
        wsum = wsum + cv * rv
    mean_v = jnp.full((L,), jnp.sum(wsum), jnp.float32) * inv_total
    inv_mean = ones / mean_v

    pltpu.sync_copy(tb_hbm.at[pl.ds(wid * OCHUNK, OCHUNK)], tb_o)
    pltpu.sync_copy(loss_hbm.at[pl.ds(wid * OCHUNK, OCHUNK)], loss_v)
    epoch0 = ep_v[pl.ds(0, L)] == 0
    alt = jnp.full((L,), 1.0 / N_CLASSES, jnp.float32)
    for k in range(OCHUNK // L):
        idx = tb_o[pl.ds(k * L, L)]
        rv = plsc.load_gather(hist, [idx])
        scale = jnp.where(epoch0, rv * inv_mean, alt)
        out_v[pl.ds(k * L, L)] = loss_v[pl.ds(k * L, L)] * scale
    pltpu.sync_copy(out_v, out_hbm.at[pl.ds(wid * OCHUNK, OCHUNK)])


_sc_call = pl.kernel(
    _body,
    out_type=jax.ShapeDtypeStruct((BS,), jnp.float32),
    mesh=plsc.VectorSubcoreMesh(core_axis_name="c", subcore_axis_name="s"),
    compiler_params=pltpu.CompilerParams(needs_layout_passes=False),
    scratch_types=[
        pltpu.VMEM((HCHUNK,), jnp.int32),
        pltpu.VMEM((NBINS,), jnp.float32),
        pltpu.VMEM((NSUB * 64,), jnp.float32),
        pltpu.VMEM((64,), jnp.float32),
        pltpu.VMEM((NBINS,), jnp.float32),
        pltpu.VMEM((OCHUNK,), jnp.int32),
        pltpu.VMEM((OCHUNK,), jnp.float32),
        pltpu.VMEM((OCHUNK,), jnp.float32),
        pltpu.VMEM((L,), jnp.int32),
        pltpu.VMEM_SHARED((NSUB * NBINS,), jnp.float32),
        pltpu.VMEM_SHARED((NBINS,), jnp.float32),
    ],
)


def kernel(loss, Tb, i_current_epoch):
    ep = jnp.broadcast_to(jnp.asarray(i_current_epoch, jnp.int32), (L,))
    return _sc_call(loss, Tb, ep)


# speedup vs baseline: 5.3378x; 1.1584x over previous
import jax
import jax.numpy as jnp
from jax import lax
from jax.experimental import pallas as pl
from jax.experimental.pallas import tpu as pltpu
from jax.experimental.pallas import tpu_sc as plsc

N_CLASSES = 1000
BS = 16384
NBINS = 1024
L = 16
NSUB = 16
NW = 32
HCHUNK = BS // NSUB
OCHUNK = BS // NW


def _body(loss_hbm, tb_hbm, ep_hbm, out_hbm,
          tb_v, hist, colblk, cnt64, cnt_v, tb_o, loss_v, out_v, ep_v,
          shared, counts_sh, sem1, sem2, sem3):
    s = lax.axis_index("s")
    c = lax.axis_index("c")
    wid = s * 2 + c

    ones = jnp.full((L,), 1.0, jnp.float32)
    zeros = jnp.zeros((L,), jnp.float32)

    d_tb = pltpu.async_copy(tb_hbm.at[pl.ds(s * HCHUNK, HCHUNK)], tb_v, sem1)
    d_to = pltpu.async_copy(tb_hbm.at[pl.ds(wid * OCHUNK, OCHUNK)], tb_o, sem2)
    d_ls = pltpu.async_copy(loss_hbm.at[pl.ds(wid * OCHUNK, OCHUNK)], loss_v, sem2)
    d_ep = pltpu.async_copy(ep_hbm, ep_v, sem2)

    for k in range(NBINS // L):
        hist[pl.ds(k * L, L)] = zeros
    d_tb.wait()
    for k in range(HCHUNK // L):
        idx = tb_v[pl.ds(k * L, L)]
        plsc.addupdate_scatter(hist, [idx], ones)
    pltpu.sync_copy(hist, shared.at[pl.ds(s * NBINS, NBINS)])
    plsc.subcore_barrier()

    fan = [pltpu.async_copy(shared.at[pl.ds(r * NBINS + s * 64, 64)],
                            colblk.at[pl.ds(r * 64, 64)], sem3)
           for r in range(NSUB)]
    for d in fan:
        d.wait()
    acc = [jnp.zeros((L,), jnp.float32) for _ in range(4)]
    for r in range(NSUB):
        for v in range(4):
            acc[v] = acc[v] + colblk[pl.ds(r * 64 + v * L, L)]
    for v in range(4):
        cnt64[pl.ds(v * L, L)] = acc[v]
    pltpu.sync_copy(cnt64, counts_sh.at[pl.ds(s * 64, 64)])
    plsc.subcore_barrier()

    pltpu.sync_copy(counts_sh, cnt_v)
    wsum = jnp.zeros((L,), jnp.float32)
    inv_total = jnp.float32(1.0 / BS)
    for k in range(NBINS // L):
        cv = cnt_v[pl.ds(k * L, L)]
        pos = cv > 0.0
        prob = jnp.where(pos, cv * inv_total, ones)
        rv = jnp.where(pos, 1.0 / prob, zeros)
        hist[pl.ds(k * L, L)] = rv
        wsum = wsum + cv * rv
    mean_v = jnp.full((L,), jnp.sum(wsum), jnp.float32) * inv_total
    inv_mean = ones / mean_v

    d_to.wait()
    d_ls.wait()
    d_ep.wait()
    epoch0 = ep_v[pl.ds(0, L)] == 0
    alt = jnp.full((L,), 1.0 / N_CLASSES, jnp.float32)
    for k in range(OCHUNK // L):
        idx = tb_o[pl.ds(k * L, L)]
        rv = plsc.load_gather(hist, [idx])
        scale = jnp.where(epoch0, rv * inv_mean, alt)
        out_v[pl.ds(k * L, L)] = loss_v[pl.ds(k * L, L)] * scale
    pltpu.sync_copy(out_v, out_hbm.at[pl.ds(wid * OCHUNK, OCHUNK)])


_sc_call = pl.kernel(
    _body,
    out_type=jax.ShapeDtypeStruct((BS,), jnp.float32),
    mesh=plsc.VectorSubcoreMesh(core_axis_name="c", subcore_axis_name="s"),
    compiler_params=pltpu.CompilerParams(needs_layout_passes=False),
    scratch_types=[
        pltpu.VMEM((HCHUNK,), jnp.int32),
        pltpu.VMEM((NBINS,), jnp.float32),
        pltpu.VMEM((NSUB * 64,), jnp.float32),
        pltpu.VMEM((64,), jnp.float32),
        pltpu.VMEM((NBINS,), jnp.float32),
        pltpu.VMEM((OCHUNK,), jnp.int32),
        pltpu.VMEM((OCHUNK,), jnp.float32),
        pltpu.VMEM((OCHUNK,), jnp.float32),
        pltpu.VMEM((L,), jnp.int32),
        pltpu.VMEM_SHARED((NSUB * NBINS,), jnp.float32),
        pltpu.VMEM_SHARED((NBINS,), jnp.float32),
        pltpu.SemaphoreType.DMA,
        pltpu.SemaphoreType.DMA,
        pltpu.SemaphoreType.DMA,
    ],
)


def kernel(loss, Tb, i_current_epoch):
    ep = jnp.broadcast_to(jnp.asarray(i_current_epoch, jnp.int32), (L,))
    return _sc_call(loss, Tb, ep)
